# baseline (speedup 1.0000x reference)
import functools

import jax
import jax.numpy as jnp
from jax import lax
from jax.experimental import pallas as pl
from jax.experimental.pallas import tpu as pltpu
from jax.experimental.pallas import tpu_sc as plsc

V = 100000
D = 128
DW = D // 2
B = 320000
NC = 2
NS = 16
NW = NC * NS
BPW = B // NW
C = 80
NCHUNK = BPW // C
L = 16
HIMASK = -65536


def _sc_body(x_st_hbm, x_vc_hbm, idx0_hbm, idx1_hbm, out_hbm,
             idx0_v, idx1_v, rows_st, rows_vc, out_v,
             sem_st0, sem_st1, sem_vc0, sem_vc1):
    wid = lax.axis_index("s") * NC + lax.axis_index("c")
    base = wid * BPW
    lane = lax.broadcasted_iota(jnp.int32, (L,), 0)
    gdn = lax.GatherDimensionNumbers(
        offset_dims=(), collapsed_slice_dims=(0,), start_index_map=(0,))

    def _perm(v, idx):
        return lax.gather(v, idx[:, None], gdn, (1,),
                          mode=lax.GatherScatterMode.PROMISE_IN_BOUNDS)

    bitrev = [0, 8, 4, 12, 2, 10, 6, 14, 1, 9, 5, 13, 3, 11, 7, 15]

    sems = ((sem_st0, sem_vc0), (sem_st1, sem_vc1))

    pltpu.sync_copy(idx0_hbm.at[pl.ds(base, BPW)], idx0_v)
    pltpu.sync_copy(idx1_hbm.at[pl.ds(base, BPW)], idx1_v)

    def fire(c, b):
        off = c * C
        pltpu.async_copy(x_st_hbm.at[idx0_v.at[pl.ds(off, C)]],
                         rows_st.at[b], sems[b][0])
        pltpu.async_copy(x_vc_hbm.at[idx1_v.at[pl.ds(off, C)]],
                         rows_vc.at[b], sems[b][1])

    def drain(b):
        pltpu.make_async_copy(x_st_hbm.at[idx0_v.at[pl.ds(0, C)]],
                              rows_st.at[b], sems[b][0]).wait()
        pltpu.make_async_copy(x_vc_hbm.at[idx1_v.at[pl.ds(0, C)]],
                              rows_vc.at[b], sems[b][1]).wait()

    def compute(c, b):
        st = rows_st.at[b]
        vc = rows_vc.at[b]

        def leaf(e):
            ps = [st[e, pl.ds(k * L, L)] * vc[e, pl.ds(k * L, L)]
                  for k in range(D // L)]
            while len(ps) > 1:
                ps = [ps[i] + ps[i + 1] for i in range(0, len(ps), 2)]
            return ps[0]

        def grp_body(g2, carry2):
            e0 = g2 * L
            vs = [leaf(e0 + r) for r in bitrev]
            for d in (8, 4, 2, 1):
                mask = (lane & d) == 0
                vs = [jnp.where(mask,
                                vs[i] + _perm(vs[i], lane ^ d),
                                vs[i + 1] + _perm(vs[i + 1], lane ^ d))
                      for i in range(0, len(vs), 2)]
            out_v[pl.ds(c * C + e0, L)] = vs[0]
            return carry2

        lax.fori_loop(0, C // L, grp_body, 0, unroll=False)

    fire(0, 0)

    def chunk_pair(g, carry):
        for i in range(2):
            c = 2 * g + i
            drain(i)
            fire(c + 1, 1 - i)
            compute(c, i)
        return carry

    lax.fori_loop(0, (NCHUNK - 1) // 2, chunk_pair, 0, unroll=False)
    drain(0)
    compute(NCHUNK - 1, 0)

    pltpu.sync_copy(out_v, out_hbm.at[pl.ds(base, BPW)])


@jax.jit
def kernel(x_st, x_vc, edge_label_index):
    idx = edge_label_index.astype(jnp.int32)
    idx0 = idx[0]
    idx1 = idx[1]

    def _pack(x):
        return x

    mesh = plsc.VectorSubcoreMesh(core_axis_name="c", subcore_axis_name="s",
                                  num_cores=NC, num_subcores=NS)
    run = pl.kernel(
        _sc_body,
        out_type=jax.ShapeDtypeStruct((B,), jnp.float32),
        mesh=mesh,
        scratch_types=[
            pltpu.VMEM((BPW,), jnp.int32),
            pltpu.VMEM((BPW,), jnp.int32),
            pltpu.VMEM((2, C, D), jnp.float32),
            pltpu.VMEM((2, C, D), jnp.float32),
            pltpu.VMEM((BPW,), jnp.float32),
            pltpu.SemaphoreType.DMA,
            pltpu.SemaphoreType.DMA,
            pltpu.SemaphoreType.DMA,
            pltpu.SemaphoreType.DMA,
        ],
    )
    return run(_pack(x_st), _pack(x_vc), idx0, idx1)


# speedup vs baseline: 1.0809x; 1.0809x over previous
import functools

import jax
import jax.numpy as jnp
from jax import lax
from jax.experimental import pallas as pl
from jax.experimental.pallas import tpu as pltpu
from jax.experimental.pallas import tpu_sc as plsc

V = 100000
D = 128
DW = D // 2
B = 320000
NC = 2
NS = 16
NW = NC * NS
BPW = B // NW
C = 80
NCHUNK = BPW // C
L = 16
HIMASK = -65536


def _sc_body(x_st_hbm, x_vc_hbm, idx0_hbm, idx1_hbm, out_hbm,
             idx0_v, idx1_v, rows_st, rows_vc, out_v,
             sem_st0, sem_st1, sem_vc0, sem_vc1):
    wid = lax.axis_index("s") * NC + lax.axis_index("c")
    base = wid * BPW
    lane = lax.broadcasted_iota(jnp.int32, (L,), 0)
    gdn = lax.GatherDimensionNumbers(
        offset_dims=(), collapsed_slice_dims=(0,), start_index_map=(0,))

    def _perm(v, idx):
        return lax.gather(v, idx[:, None], gdn, (1,),
                          mode=lax.GatherScatterMode.PROMISE_IN_BOUNDS)

    bitrev = [0, 8, 4, 12, 2, 10, 6, 14, 1, 9, 5, 13, 3, 11, 7, 15]

    sems = ((sem_st0, sem_vc0), (sem_st1, sem_vc1))

    pltpu.sync_copy(idx0_hbm.at[pl.ds(base, BPW)], idx0_v)
    pltpu.sync_copy(idx1_hbm.at[pl.ds(base, BPW)], idx1_v)

    def fire(c, b):
        off = c * C
        pltpu.async_copy(x_st_hbm.at[idx0_v.at[pl.ds(off, C)]],
                         rows_st.at[b], sems[b][0])
        pltpu.async_copy(x_vc_hbm.at[idx1_v.at[pl.ds(off, C)]],
                         rows_vc.at[b], sems[b][1])

    def drain(b):
        pltpu.make_async_copy(x_st_hbm.at[idx0_v.at[pl.ds(0, C)]],
                              rows_st.at[b], sems[b][0]).wait()
        pltpu.make_async_copy(x_vc_hbm.at[idx1_v.at[pl.ds(0, C)]],
                              rows_vc.at[b], sems[b][1]).wait()

    def compute(c, b):
        st = rows_st.at[b]
        vc = rows_vc.at[b]

        def leaf(e):
            ps = [st[e, pl.ds(k * L, L)] * vc[e, pl.ds(k * L, L)]
                  for k in range(D // L)]
            while len(ps) > 1:
                ps = [ps[i] + ps[i + 1] for i in range(0, len(ps), 2)]
            return ps[0]

        masks = {d: (lane & d) == 0 for d in (8, 4, 2, 1)}
        perms = {d: lane ^ d for d in (8, 4, 2, 1)}

        def subtree(e0, i, level):
            if level == 0:
                return leaf(e0 + bitrev[i])
            half = 1 << (level - 1)
            u = subtree(e0, i, level - 1)
            v = subtree(e0, i + half, level - 1)
            d = L >> level
            return jnp.where(masks[d],
                             u + _perm(u, perms[d]),
                             v + _perm(v, perms[d]))

        def grp_body(g2, carry2):
            e0 = g2 * L
            out_v[pl.ds(c * C + e0, L)] = subtree(e0, 0, 4)
            return carry2

        lax.fori_loop(0, C // L, grp_body, 0, unroll=False)

    fire(0, 0)

    def chunk_pair(g, carry):
        for i in range(2):
            c = 2 * g + i
            drain(i)
            fire(c + 1, 1 - i)
            compute(c, i)
        return carry

    lax.fori_loop(0, (NCHUNK - 1) // 2, chunk_pair, 0, unroll=False)
    drain(0)
    compute(NCHUNK - 1, 0)

    pltpu.sync_copy(out_v, out_hbm.at[pl.ds(base, BPW)])


@jax.jit
def kernel(x_st, x_vc, edge_label_index):
    idx = edge_label_index.astype(jnp.int32)
    idx0 = idx[0]
    idx1 = idx[1]

    def _pack(x):
        return x

    mesh = plsc.VectorSubcoreMesh(core_axis_name="c", subcore_axis_name="s",
                                  num_cores=NC, num_subcores=NS)
    run = pl.kernel(
        _sc_body,
        out_type=jax.ShapeDtypeStruct((B,), jnp.float32),
        mesh=mesh,
        scratch_types=[
            pltpu.VMEM((BPW,), jnp.int32),
            pltpu.VMEM((BPW,), jnp.int32),
            pltpu.VMEM((2, C, D), jnp.float32),
            pltpu.VMEM((2, C, D), jnp.float32),
            pltpu.VMEM((BPW,), jnp.float32),
            pltpu.SemaphoreType.DMA,
            pltpu.SemaphoreType.DMA,
            pltpu.SemaphoreType.DMA,
            pltpu.SemaphoreType.DMA,
        ],
    )
    return run(_pack(x_st), _pack(x_vc), idx0, idx1)


# speedup vs baseline: 1.8538x; 1.7151x over previous
import functools

import jax
import jax.numpy as jnp
from jax import lax
from jax.experimental import pallas as pl
from jax.experimental.pallas import tpu as pltpu
from jax.experimental.pallas import tpu_sc as plsc

V = 100000
D = 128
DW = D // 2
B = 320000
NC = 2
NS = 16
NW = NC * NS
BPW = B // NW
C = 80
NCHUNK = BPW // C
L = 16
HIMASK = -65536


def _sc_body(x_st_hbm, x_vc_hbm, idx0_hbm, idx1_hbm, out_hbm,
             idx0_v, idx1_v, rows_st, rows_vc, out_v,
             sem_st0, sem_st1, sem_vc0, sem_vc1):
    wid = lax.axis_index("s") * NC + lax.axis_index("c")
    base = wid * BPW
    lane = lax.broadcasted_iota(jnp.int32, (L,), 0)
    gdn = lax.GatherDimensionNumbers(
        offset_dims=(), collapsed_slice_dims=(0,), start_index_map=(0,))

    def _perm(v, idx):
        return lax.gather(v, idx[:, None], gdn, (1,),
                          mode=lax.GatherScatterMode.PROMISE_IN_BOUNDS)

    bfly = [lane ^ d for d in (8, 4, 2, 1)]
    mask0 = lane == 0

    def _lane_sum(v):
        for idx in bfly:
            v = v + _perm(v, idx)
        return v

    sems = ((sem_st0, sem_vc0), (sem_st1, sem_vc1))

    pltpu.sync_copy(idx0_hbm.at[pl.ds(base, BPW)], idx0_v)
    pltpu.sync_copy(idx1_hbm.at[pl.ds(base, BPW)], idx1_v)

    def fire(c, b):
        off = c * C
        pltpu.async_copy(x_st_hbm.at[idx0_v.at[pl.ds(off, C)]],
                         rows_st.at[b], sems[b][0])
        pltpu.async_copy(x_vc_hbm.at[idx1_v.at[pl.ds(off, C)]],
                         rows_vc.at[b], sems[b][1])

    def drain(b):
        pltpu.make_async_copy(x_st_hbm.at[idx0_v.at[pl.ds(0, C)]],
                              rows_st.at[b], sems[b][0]).wait()
        pltpu.make_async_copy(x_vc_hbm.at[idx1_v.at[pl.ds(0, C)]],
                              rows_vc.at[b], sems[b][1]).wait()

    def compute(c, b):
        st = rows_st.at[b]
        vc = rows_vc.at[b]

        def leaf(e):
            ps = [st[e, pl.ds(k * L, L)] * vc[e, pl.ds(k * L, L)]
                  for k in range(D // L)]
            while len(ps) > 1:
                ps = [ps[i] + ps[i + 1] for i in range(0, len(ps), 2)]
            return ps[0]

        def grp_body(g2, carry2):
            e0 = g2 * L

            def edge_body(j, res):
                v = _lane_sum(leaf(e0 + j))
                return jnp.where(lane == j, v, res)

            res = lax.fori_loop(0, L, edge_body,
                                jnp.zeros((L,), jnp.float32), unroll=2)
            out_v[pl.ds(c * C + e0, L)] = res
            return carry2

        lax.fori_loop(0, C // L, grp_body, 0, unroll=False)

    fire(0, 0)

    def chunk_pair(g, carry):
        for i in range(2):
            c = 2 * g + i
            drain(i)
            fire(c + 1, 1 - i)
            compute(c, i)
        return carry

    lax.fori_loop(0, (NCHUNK - 1) // 2, chunk_pair, 0, unroll=False)
    drain(0)
    compute(NCHUNK - 1, 0)

    pltpu.sync_copy(out_v, out_hbm.at[pl.ds(base, BPW)])


@jax.jit
def kernel(x_st, x_vc, edge_label_index):
    idx = edge_label_index.astype(jnp.int32)
    idx0 = idx[0]
    idx1 = idx[1]

    def _pack(x):
        return x

    mesh = plsc.VectorSubcoreMesh(core_axis_name="c", subcore_axis_name="s",
                                  num_cores=NC, num_subcores=NS)
    run = pl.kernel(
        _sc_body,
        out_type=jax.ShapeDtypeStruct((B,), jnp.float32),
        mesh=mesh,
        scratch_types=[
            pltpu.VMEM((BPW,), jnp.int32),
            pltpu.VMEM((BPW,), jnp.int32),
            pltpu.VMEM((2, C, D), jnp.float32),
            pltpu.VMEM((2, C, D), jnp.float32),
            pltpu.VMEM((BPW,), jnp.float32),
            pltpu.SemaphoreType.DMA,
            pltpu.SemaphoreType.DMA,
            pltpu.SemaphoreType.DMA,
            pltpu.SemaphoreType.DMA,
        ],
    )
    return run(_pack(x_st), _pack(x_vc), idx0, idx1)


# speedup vs baseline: 2.0471x; 1.1043x over previous
import functools

import jax
import jax.numpy as jnp
from jax import lax
from jax.experimental import pallas as pl
from jax.experimental.pallas import tpu as pltpu
from jax.experimental.pallas import tpu_sc as plsc

V = 100000
D = 128
B = 320000
NC = 2
NS = 16
NW = NC * NS
BPW = B // NW
C = 128
NFULL = BPW // C
TAIL = BPW - NFULL * C
L = 16


def _sc_body(x_st_hbm, x_vc_hbm, idx0_hbm, idx1_hbm, out_hbm,
             idx0_v, idx1_v, rows_st, rows_vc, out_v,
             sem_st0, sem_st1, sem_vc0, sem_vc1):
    wid = lax.axis_index("s") * NC + lax.axis_index("c")
    base = wid * BPW
    lane = lax.broadcasted_iota(jnp.int32, (L,), 0)
    gdn = lax.GatherDimensionNumbers(
        offset_dims=(), collapsed_slice_dims=(0,), start_index_map=(0,))

    def _perm(v, idx):
        return lax.gather(v, idx[:, None], gdn, (1,),
                          mode=lax.GatherScatterMode.PROMISE_IN_BOUNDS)

    bfly = [lane ^ d for d in (8, 4, 2, 1)]

    def _lane_sum(v):
        for idx in bfly:
            v = v + _perm(v, idx)
        return v

    sems = ((sem_st0, sem_vc0), (sem_st1, sem_vc1))

    pltpu.sync_copy(idx0_hbm.at[pl.ds(base, BPW)], idx0_v)
    pltpu.sync_copy(idx1_hbm.at[pl.ds(base, BPW)], idx1_v)

    def fire(c, b, n=C):
        off = c * C
        pltpu.async_copy(x_st_hbm.at[idx0_v.at[pl.ds(off, n)]],
                         rows_st.at[b, pl.ds(0, n)], sems[b][0])
        pltpu.async_copy(x_vc_hbm.at[idx1_v.at[pl.ds(off, n)]],
                         rows_vc.at[b, pl.ds(0, n)], sems[b][1])

    def drain(b, n=C):
        pltpu.make_async_copy(x_st_hbm.at[idx0_v.at[pl.ds(0, n)]],
                              rows_st.at[b, pl.ds(0, n)], sems[b][0]).wait()
        pltpu.make_async_copy(x_vc_hbm.at[idx1_v.at[pl.ds(0, n)]],
                              rows_vc.at[b, pl.ds(0, n)], sems[b][1]).wait()

    def compute(c, b, ngrp=C // L):
        st = rows_st.at[b]
        vc = rows_vc.at[b]

        def leaf(e):
            ps = [st[e, pl.ds(k * L, L)] * vc[e, pl.ds(k * L, L)]
                  for k in range(D // L)]
            while len(ps) > 1:
                ps = [ps[i] + ps[i + 1] for i in range(0, len(ps), 2)]
            return ps[0]

        def grp_body(g2, carry2):
            e0 = g2 * L

            def edge_body(j, res):
                v = _lane_sum(leaf(e0 + j))
                return jnp.where(lane == j, v, res)

            res = lax.fori_loop(0, L, edge_body,
                                jnp.zeros((L,), jnp.float32), unroll=2)
            out_v[pl.ds(c * C + e0, L)] = res
            return carry2

        lax.fori_loop(0, ngrp, grp_body, 0, unroll=False)

    fire(0, 0)

    def chunk_pair(g, carry):
        for i in range(2):
            c = 2 * g + i
            drain(i)
            fire(c + 1, 1 - i)
            compute(c, i)
        return carry

    lax.fori_loop(0, (NFULL - 2) // 2, chunk_pair, 0, unroll=False)
    drain(0)
    fire(NFULL - 1, 1)
    compute(NFULL - 2, 0)
    drain(1)
    fire(NFULL, 0, TAIL)
    compute(NFULL - 1, 1)
    drain(0, TAIL)
    compute(NFULL, 0, TAIL // L)

    pltpu.sync_copy(out_v, out_hbm.at[pl.ds(base, BPW)])


@jax.jit
def kernel(x_st, x_vc, edge_label_index):
    idx = edge_label_index.astype(jnp.int32)
    idx0 = idx[0]
    idx1 = idx[1]

    mesh = plsc.VectorSubcoreMesh(core_axis_name="c", subcore_axis_name="s",
                                  num_cores=NC, num_subcores=NS)
    run = pl.kernel(
        _sc_body,
        out_type=jax.ShapeDtypeStruct((B,), jnp.float32),
        mesh=mesh,
        scratch_types=[
            pltpu.VMEM((BPW,), jnp.int32),
            pltpu.VMEM((BPW,), jnp.int32),
            pltpu.VMEM((2, C, D), jnp.float32),
            pltpu.VMEM((2, C, D), jnp.float32),
            pltpu.VMEM((BPW,), jnp.float32),
            pltpu.SemaphoreType.DMA,
            pltpu.SemaphoreType.DMA,
            pltpu.SemaphoreType.DMA,
            pltpu.SemaphoreType.DMA,
        ],
    )
    return run(x_st, x_vc, idx0, idx1)


# speedup vs baseline: 2.7164x; 1.3269x over previous
import functools

import jax
import jax.numpy as jnp
from jax import lax
from jax.experimental import pallas as pl
from jax.experimental.pallas import tpu as pltpu
from jax.experimental.pallas import tpu_sc as plsc

V = 100000
D = 128
B = 320000
NC = 2
NS = 16
NW = NC * NS
BPW = B // NW
C = 128
NFULL = BPW // C
TAIL = BPW - NFULL * C
L = 16


def _sc_body(x_st_hbm, x_vc_hbm, idx0_hbm, idx1_hbm, out_hbm,
             idx0_v, idx1_v, rows_st, rows_vc, out_v,
             sem_st0, sem_st1, sem_st2, sem_vc0, sem_vc1, sem_vc2):
    wid = lax.axis_index("s") * NC + lax.axis_index("c")
    base = wid * BPW
    lane = lax.broadcasted_iota(jnp.int32, (L,), 0)
    gdn = lax.GatherDimensionNumbers(
        offset_dims=(), collapsed_slice_dims=(0,), start_index_map=(0,))

    def _perm(v, idx):
        return lax.gather(v, idx[:, None], gdn, (1,),
                          mode=lax.GatherScatterMode.PROMISE_IN_BOUNDS)

    bfly = [lane ^ d for d in (8, 4, 2, 1)]

    def _lane_sum(v):
        for idx in bfly:
            v = v + _perm(v, idx)
        return v

    sems = ((sem_st0, sem_vc0), (sem_st1, sem_vc1), (sem_st2, sem_vc2))

    pltpu.sync_copy(idx0_hbm.at[pl.ds(base, BPW)], idx0_v)
    pltpu.sync_copy(idx1_hbm.at[pl.ds(base, BPW)], idx1_v)

    def fire(c, b, n=C):
        off = c * C
        pltpu.async_copy(x_st_hbm.at[idx0_v.at[pl.ds(off, n)]],
                         rows_st.at[b, pl.ds(0, n)], sems[b][0])
        pltpu.async_copy(x_vc_hbm.at[idx1_v.at[pl.ds(off, n)]],
                         rows_vc.at[b, pl.ds(0, n)], sems[b][1])

    def drain(b, n=C):
        pltpu.make_async_copy(x_st_hbm.at[idx0_v.at[pl.ds(0, n)]],
                              rows_st.at[b, pl.ds(0, n)], sems[b][0]).wait()
        pltpu.make_async_copy(x_vc_hbm.at[idx1_v.at[pl.ds(0, n)]],
                              rows_vc.at[b, pl.ds(0, n)], sems[b][1]).wait()

    def compute(c, b, ngrp=C // L):
        st = rows_st.at[b]
        vc = rows_vc.at[b]

        def leaf(e):
            ps = [st[e, pl.ds(k * L, L)] * vc[e, pl.ds(k * L, L)]
                  for k in range(D // L)]
            while len(ps) > 1:
                ps = [ps[i] + ps[i + 1] for i in range(0, len(ps), 2)]
            return ps[0]

        def grp_body(g2, carry2):
            e0 = g2 * L

            def edge_body(j, res):
                v = _lane_sum(leaf(e0 + j))
                return jnp.where(lane == j, v, res)

            res = lax.fori_loop(0, L, edge_body,
                                jnp.zeros((L,), jnp.float32), unroll=2)
            out_v[pl.ds(c * C + e0, L)] = res
            return carry2

        lax.fori_loop(0, ngrp, grp_body, 0, unroll=False)

    fire(0, 0)
    fire(1, 1)

    def chunk_triple(g, carry):
        for i in range(3):
            c = 3 * g + i
            drain(i)
            fire(c + 2, (i + 2) % 3)
            compute(c, i)
        return carry

    lax.fori_loop(0, (NFULL - 3) // 3, chunk_triple, 0, unroll=False)
    drain(0)
    fire(NFULL - 1, 2)
    compute(NFULL - 3, 0)
    drain(1)
    fire(NFULL, 0, TAIL)
    compute(NFULL - 2, 1)
    drain(2)
    compute(NFULL - 1, 2)
    drain(0, TAIL)
    compute(NFULL, 0, TAIL // L)

    pltpu.sync_copy(out_v, out_hbm.at[pl.ds(base, BPW)])


@jax.jit
def kernel(x_st, x_vc, edge_label_index):
    idx = edge_label_index.astype(jnp.int32)
    idx0 = idx[0]
    idx1 = idx[1]

    mesh = plsc.VectorSubcoreMesh(core_axis_name="c", subcore_axis_name="s",
                                  num_cores=NC, num_subcores=NS)
    run = pl.kernel(
        _sc_body,
        out_type=jax.ShapeDtypeStruct((B,), jnp.float32),
        mesh=mesh,
        scratch_types=[
            pltpu.VMEM((BPW,), jnp.int32),
            pltpu.VMEM((BPW,), jnp.int32),
            pltpu.VMEM((3, C, D), jnp.float32),
            pltpu.VMEM((3, C, D), jnp.float32),
            pltpu.VMEM((BPW,), jnp.float32),
            pltpu.SemaphoreType.DMA,
            pltpu.SemaphoreType.DMA,
            pltpu.SemaphoreType.DMA,
            pltpu.SemaphoreType.DMA,
            pltpu.SemaphoreType.DMA,
            pltpu.SemaphoreType.DMA,
        ],
    )
    return run(x_st, x_vc, idx0, idx1)


# speedup vs baseline: 2.7321x; 1.0058x over previous
import functools

import jax
import jax.numpy as jnp
from jax import lax
from jax.experimental import pallas as pl
from jax.experimental.pallas import tpu as pltpu
from jax.experimental.pallas import tpu_sc as plsc

V = 100000
D = 128
B = 320000
NC = 2
NS = 16
NW = NC * NS
BPW = B // NW
C = 96
NFULL = BPW // C
TAIL = BPW - NFULL * C
L = 16


def _sc_body(x_st_hbm, x_vc_hbm, idx0_hbm, idx1_hbm, out_hbm,
             idx0_v, idx1_v, rows_st, rows_vc, out_v,
             sem_st0, sem_st1, sem_st2, sem_st3,
             sem_vc0, sem_vc1, sem_vc2, sem_vc3):
    wid = lax.axis_index("s") * NC + lax.axis_index("c")
    base = wid * BPW
    lane = lax.broadcasted_iota(jnp.int32, (L,), 0)
    gdn = lax.GatherDimensionNumbers(
        offset_dims=(), collapsed_slice_dims=(0,), start_index_map=(0,))

    def _perm(v, idx):
        return lax.gather(v, idx[:, None], gdn, (1,),
                          mode=lax.GatherScatterMode.PROMISE_IN_BOUNDS)

    bfly = [lane ^ d for d in (8, 4, 2, 1)]

    def _lane_sum(v):
        for idx in bfly:
            v = v + _perm(v, idx)
        return v

    sems = ((sem_st0, sem_vc0), (sem_st1, sem_vc1),
            (sem_st2, sem_vc2), (sem_st3, sem_vc3))

    pltpu.sync_copy(idx0_hbm.at[pl.ds(base, BPW)], idx0_v)
    pltpu.sync_copy(idx1_hbm.at[pl.ds(base, BPW)], idx1_v)

    def fire(c, b, n=C):
        off = c * C
        pltpu.async_copy(x_st_hbm.at[idx0_v.at[pl.ds(off, n)]],
                         rows_st.at[b, pl.ds(0, n)], sems[b][0])
        pltpu.async_copy(x_vc_hbm.at[idx1_v.at[pl.ds(off, n)]],
                         rows_vc.at[b, pl.ds(0, n)], sems[b][1])

    def drain(b, n=C):
        pltpu.make_async_copy(x_st_hbm.at[idx0_v.at[pl.ds(0, n)]],
                              rows_st.at[b, pl.ds(0, n)], sems[b][0]).wait()
        pltpu.make_async_copy(x_vc_hbm.at[idx1_v.at[pl.ds(0, n)]],
                              rows_vc.at[b, pl.ds(0, n)], sems[b][1]).wait()

    def compute(c, b, ngrp=C // L):
        st = rows_st.at[b]
        vc = rows_vc.at[b]

        def leaf(e):
            ps = [st[e, pl.ds(k * L, L)] * vc[e, pl.ds(k * L, L)]
                  for k in range(D // L)]
            while len(ps) > 1:
                ps = [ps[i] + ps[i + 1] for i in range(0, len(ps), 2)]
            return ps[0]

        def grp_body(g2, carry2):
            e0 = g2 * L

            def edge_body(j, res):
                v = _lane_sum(leaf(e0 + j))
                return jnp.where(lane == j, v, res)

            res = lax.fori_loop(0, L, edge_body,
                                jnp.zeros((L,), jnp.float32), unroll=2)
            out_v[pl.ds(c * C + e0, L)] = res
            return carry2

        lax.fori_loop(0, ngrp, grp_body, 0, unroll=False)

    fire(0, 0)
    fire(1, 1)
    fire(2, 2)

    def chunk_quad(g, carry):
        for i in range(4):
            c = 4 * g + i
            drain(i)
            fire(c + 3, (i + 3) % 4)
            compute(c, i)
        return carry

    lax.fori_loop(0, (NFULL - 4) // 4, chunk_quad, 0, unroll=False)
    drain(0)
    fire(NFULL - 1, 3)
    compute(NFULL - 4, 0)
    drain(1)
    fire(NFULL, 0, TAIL)
    compute(NFULL - 3, 1)
    drain(2)
    compute(NFULL - 2, 2)
    drain(3)
    compute(NFULL - 1, 3)
    drain(0, TAIL)
    compute(NFULL, 0, TAIL // L)

    pltpu.sync_copy(out_v, out_hbm.at[pl.ds(base, BPW)])


@jax.jit
def kernel(x_st, x_vc, edge_label_index):
    idx = edge_label_index.astype(jnp.int32)
    idx0 = idx[0]
    idx1 = idx[1]

    mesh = plsc.VectorSubcoreMesh(core_axis_name="c", subcore_axis_name="s",
                                  num_cores=NC, num_subcores=NS)
    run = pl.kernel(
        _sc_body,
        out_type=jax.ShapeDtypeStruct((B,), jnp.float32),
        mesh=mesh,
        scratch_types=[
            pltpu.VMEM((BPW,), jnp.int32),
            pltpu.VMEM((BPW,), jnp.int32),
            pltpu.VMEM((4, C, D), jnp.float32),
            pltpu.VMEM((4, C, D), jnp.float32),
            pltpu.VMEM((BPW,), jnp.float32),
            pltpu.SemaphoreType.DMA,
            pltpu.SemaphoreType.DMA,
            pltpu.SemaphoreType.DMA,
            pltpu.SemaphoreType.DMA,
            pltpu.SemaphoreType.DMA,
            pltpu.SemaphoreType.DMA,
            pltpu.SemaphoreType.DMA,
            pltpu.SemaphoreType.DMA,
        ],
    )
    return run(x_st, x_vc, idx0, idx1)
